# baseline (speedup 1.0000x reference)
import functools

import jax
import jax.numpy as jnp
from jax import lax
from jax.experimental import pallas as pl
from jax.experimental.pallas import tpu as pltpu
from jax.experimental.pallas import tpu_sc as plsc

BATCH = 16384
D_FEAT = 128
EMBED_SIZE = 64

NC = 2
NS = 16
NW = NC * NS
B_PER_W = BATCH // NW
IDX_CHUNK = 128
N_CHUNK = B_PER_W // IDX_CHUNK

_SC_MESH = plsc.VectorSubcoreMesh(core_axis_name="c", subcore_axis_name="s",
                                  num_cores=NC, num_subcores=NS)


def _make_row_gather(width):
    def body(ids_hbm, tab_hbm, out_hbm, idx_v, rows_v, sem):
        wid = lax.axis_index("s") * NC + lax.axis_index("c")
        base = wid * B_PER_W
        pltpu.sync_copy(ids_hbm.at[wid], idx_v)
        for j in range(N_CHUNK):
            pltpu.async_copy(tab_hbm.at[idx_v.at[j]],
                             rows_v.at[pl.ds(j * IDX_CHUNK, IDX_CHUNK)], sem)
        for j in range(N_CHUNK):
            pltpu.make_async_copy(
                tab_hbm.at[idx_v.at[j]],
                rows_v.at[pl.ds(j * IDX_CHUNK, IDX_CHUNK)], sem).wait()
        pltpu.sync_copy(rows_v, out_hbm.at[pl.ds(base, B_PER_W)])

    return pl.kernel(
        body,
        out_type=jax.ShapeDtypeStruct((BATCH, width), jnp.float32),
        mesh=_SC_MESH,
        compiler_params=pltpu.CompilerParams(use_tc_tiling_on_sc=False),
        scratch_types=[
            pltpu.VMEM((N_CHUNK, IDX_CHUNK), jnp.int32),
            pltpu.VMEM((B_PER_W, width), jnp.float32),
            pltpu.SemaphoreType.DMA,
        ],
    )


_gather_feat = _make_row_gather(D_FEAT)
_gather_pairs = _make_row_gather(2 * EMBED_SIZE)


def _proj_body(x_ref, w_ref, b_ref, o_ref):
    o_ref[...] = (jnp.dot(w_ref[...], x_ref[...].T,
                          preferred_element_type=jnp.float32) + b_ref[...])


_ROWS_PER_BLK = 2048


def _tc_proj(x, w, b2d):
    return pl.pallas_call(
        _proj_body,
        grid=(BATCH // _ROWS_PER_BLK,),
        in_specs=[
            pl.BlockSpec((_ROWS_PER_BLK, D_FEAT), lambda i: (i, 0)),
            pl.BlockSpec((EMBED_SIZE, D_FEAT), lambda i: (0, 0)),
            pl.BlockSpec((EMBED_SIZE, 1), lambda i: (0, 0)),
        ],
        out_specs=pl.BlockSpec((EMBED_SIZE, _ROWS_PER_BLK), lambda i: (0, i)),
        out_shape=jax.ShapeDtypeStruct((EMBED_SIZE, BATCH), jnp.float32),
    )(x, w, b2d)


def _half_body(x_ref, p_ref, o_ref):
    lo = x_ref[:, :EMBED_SIZE]
    hi = x_ref[:, EMBED_SIZE:]
    o_ref[...] = lo + p_ref[...] * (hi - lo)


def _tc_half(pairs, parity):
    return pl.pallas_call(
        _half_body,
        grid=(BATCH // _ROWS_PER_BLK,),
        in_specs=[
            pl.BlockSpec((_ROWS_PER_BLK, 2 * EMBED_SIZE), lambda i: (i, 0)),
            pl.BlockSpec((_ROWS_PER_BLK, 1), lambda i: (i, 0)),
        ],
        out_specs=pl.BlockSpec((_ROWS_PER_BLK, EMBED_SIZE), lambda i: (i, 0)),
        out_shape=jax.ShapeDtypeStruct((BATCH, EMBED_SIZE), jnp.float32),
    )(pairs, parity)


def kernel(node_ids_feat, node_ids_embed, feat_table, proj_W, proj_b,
           embed_table):
    ids_f = node_ids_feat.astype(jnp.int32).reshape(NW, N_CHUNK, IDX_CHUNK)
    ids_e = node_ids_embed.astype(jnp.int32)
    ids_pair = (ids_e >> 1).reshape(NW, N_CHUNK, IDX_CHUNK)
    parity = (ids_e & 1).astype(jnp.float32).reshape(BATCH, 1)
    lin128 = embed_table.reshape(embed_table.shape[0] // 2, 2 * EMBED_SIZE)
    pairs = _gather_pairs(ids_pair, lin128)
    gathered = _gather_feat(ids_f, feat_table)
    emb_embed = _tc_half(pairs, parity)
    feat_T = _tc_proj(gathered, proj_W, proj_b.reshape(EMBED_SIZE, 1))
    return (feat_T.T, emb_embed)
